# baseline (speedup 1.0000x reference)
import functools
import math

import jax
import jax.numpy as jnp
from jax.experimental import pallas as pl
from jax.experimental.pallas import tpu as pltpu

_BT = 256


def _tc1_body(hs_ref, q_ref, rwt_ref, inwt_ref, inb_ref,
              mqs_ref, ids_ref, tw_ref, cnt_ref, disp_ref, prob_ref,
              loss_ref, *, n_tokens, n_experts, scale):
    i = pl.program_id(0)
    nsteps = pl.num_programs(0)
    T = hs_ref.shape[0]
    E = n_experts

    logits = jnp.dot(hs_ref[...].astype(jnp.bfloat16),
                     rwt_ref[...].astype(jnp.bfloat16),
                     preferred_element_type=jnp.float32)
    w = jax.nn.softmax(logits, axis=-1)
    eids = jax.lax.broadcasted_iota(jnp.int32, (T, E), 1)
    w1 = jnp.max(w, axis=-1, keepdims=True)
    i1 = jnp.argmax(w, axis=-1).reshape(T, 1)
    wm = jnp.where(eids == i1, -jnp.inf, w)
    w2 = jnp.max(wm, axis=-1, keepdims=True)
    i2 = jnp.argmax(wm, axis=-1).reshape(T, 1)
    denom = w1 + w2 + 1e-8
    ids_ref[...] = jnp.concatenate([i1, i2], axis=1)
    tw_ref[...] = jnp.concatenate([w1 / denom, w2 / denom], axis=1)

    m1 = (eids == i1).astype(jnp.float32)
    m2 = (eids == i2).astype(jnp.float32)
    mboth = m1 + m2
    sub = [jnp.sum(mboth[c * 256:(c + 1) * 256], axis=0, keepdims=True)
           for c in range(T // 256)]
    cnt_ref[...] = jnp.concatenate(sub, axis=0).astype(
        jnp.int32).reshape(cnt_ref.shape)

    disp_part = jnp.sum(m1, axis=0, keepdims=True)
    prob_part = jnp.sum(w, axis=0, keepdims=True)

    @pl.when(i == 0)
    def _init():
        disp_ref[...] = jnp.zeros_like(disp_ref)
        prob_ref[...] = jnp.zeros_like(prob_ref)

    disp_ref[...] += disp_part
    prob_ref[...] += prob_part

    @pl.when(i == nsteps - 1)
    def _finalize():
        df = disp_ref[...] / n_tokens
        pf = prob_ref[...] / n_tokens
        loss_ref[...] = (E * jnp.sum(df * pf)).reshape(1, 1)

    log2e = 1.4426950408889634
    mq = jnp.dot(q_ref[...].astype(jnp.bfloat16),
                 inwt_ref[...].astype(jnp.bfloat16),
                 preferred_element_type=jnp.float32) + inb_ref[...]
    mqs_ref[...] = mq * (scale * log2e)


def _tc2_body(blocke_sref, mqp_ref, wrow_ref, mem_ref, memt_ref, outp_ref):
    b = pl.program_id(0)
    e = blocke_sref[b]
    mqb = mqp_ref[...].astype(jnp.bfloat16)
    attn = jnp.dot(mqb, memt_ref[e].astype(jnp.bfloat16),
                   preferred_element_type=jnp.float32)
    p = jnp.exp2(attn)
    s = jnp.sum(p, axis=-1, keepdims=True)
    eo = jnp.dot(p.astype(jnp.bfloat16), mem_ref[e].astype(jnp.bfloat16),
                 preferred_element_type=jnp.float32)
    wcol = wrow_ref[0, 0, :].reshape(outp_ref.shape[0], 1)
    outp_ref[...] = eo * (wcol / s)


def _tc3_body(c0_ref, c1_ref, outwt_ref, outb_ref, out_ref):
    comb = (c0_ref[...] + c1_ref[...]).astype(jnp.bfloat16)
    out_ref[...] = jnp.dot(comb, outwt_ref[...].astype(jnp.bfloat16),
                           preferred_element_type=jnp.float32) + outb_ref[...]


def kernel(hidden_states, query, router_W, in_W, in_b, out_W, out_b, memory):
    B, S, D = hidden_states.shape
    E = router_W.shape[0]
    DM = in_W.shape[0]
    C = memory.shape[1]
    N = B * S
    P = 2 * N
    BT = _BT
    NB = P // BT + E
    NPAD = NB * BT
    T1 = min(1024, N)
    scale = 1.0 / math.sqrt(DM)

    hs2 = hidden_states.reshape(N, D)
    q2 = query.reshape(N, D)
    rwt = router_W.T
    inwt = in_W.T
    outwt = out_W.T
    memt = memory.transpose(0, 2, 1)
    inb2 = in_b.reshape(1, DM)
    outb2 = out_b.reshape(1, D)

    tc1 = functools.partial(_tc1_body, n_tokens=float(N), n_experts=E,
                            scale=scale)
    mqs, ids, tw, cnts, _, _, loss = pl.pallas_call(
        tc1,
        grid=(N // T1,),
        in_specs=[
            pl.BlockSpec((T1, D), lambda i: (i, 0)),
            pl.BlockSpec((T1, D), lambda i: (i, 0)),
            pl.BlockSpec((D, E), lambda i: (0, 0)),
            pl.BlockSpec((D, DM), lambda i: (0, 0)),
            pl.BlockSpec((1, DM), lambda i: (0, 0)),
        ],
        out_specs=[
            pl.BlockSpec((T1, DM), lambda i: (i, 0)),
            pl.BlockSpec((T1, 2), lambda i: (i, 0)),
            pl.BlockSpec((T1, 2), lambda i: (i, 0)),
            pl.BlockSpec((1, T1 // 256, E), lambda i: (i, 0, 0)),
            pl.BlockSpec((1, E), lambda i: (0, 0)),
            pl.BlockSpec((1, E), lambda i: (0, 0)),
            pl.BlockSpec((1, 1), lambda i: (0, 0)),
        ],
        out_shape=[
            jax.ShapeDtypeStruct((N, DM), jnp.float32),
            jax.ShapeDtypeStruct((N, 2), jnp.int32),
            jax.ShapeDtypeStruct((N, 2), jnp.float32),
            jax.ShapeDtypeStruct((N // T1, T1 // 256, E), jnp.int32),
            jax.ShapeDtypeStruct((1, E), jnp.float32),
            jax.ShapeDtypeStruct((1, E), jnp.float32),
            jax.ShapeDtypeStruct((1, 1), jnp.float32),
        ],
    )(hs2, q2, rwt, inwt, inb2)

    ids_flat = ids.reshape(P)
    tw_flat = tw.reshape(P)
    total = jnp.sum(cnts, axis=(0, 1))
    aligned = ((total + BT - 1) // BT) * BT
    base = jnp.cumsum(aligned) - aligned
    order = jnp.argsort(ids_flat, stable=True)
    gstart = jnp.cumsum(total) - total
    sorted_e = ids_flat[order]
    q_idx = jnp.arange(P, dtype=jnp.int32)
    slot_sorted = base[sorted_e] + (q_idx - gstart[sorted_e])
    slot = jnp.zeros((P,), jnp.int32).at[order].set(slot_sorted.astype(jnp.int32))
    perm_tok = jnp.zeros((NPAD,), jnp.int32).at[slot].set(
        (q_idx >> 1).astype(jnp.int32))
    perm_w = jnp.zeros((NPAD,), jnp.float32).at[slot].set(tw_flat)
    endblk = (base + aligned) // BT
    bidx = jnp.arange(NB, dtype=jnp.int32)
    block_e = jnp.minimum(
        jnp.sum((bidx[:, None] >= endblk[None, :]).astype(jnp.int32), axis=1),
        E - 1).astype(jnp.int32)
    pos = slot.reshape(N, 2).T
    mq_perm = mqs[perm_tok]

    wrow = perm_w.reshape(NB, 1, BT)
    grid_spec = pltpu.PrefetchScalarGridSpec(
        num_scalar_prefetch=1,
        grid=(NB,),
        in_specs=[
            pl.BlockSpec((BT, DM), lambda b, sref: (b, 0)),
            pl.BlockSpec((1, 1, BT), lambda b, sref: (b, 0, 0)),
            pl.BlockSpec((E, C, DM), lambda b, sref: (0, 0, 0)),
            pl.BlockSpec((E, DM, C), lambda b, sref: (0, 0, 0)),
        ],
        out_specs=pl.BlockSpec((BT, DM), lambda b, sref: (b, 0)),
    )
    out_perm = pl.pallas_call(
        _tc2_body,
        grid_spec=grid_spec,
        out_shape=jax.ShapeDtypeStruct((NPAD, DM), jnp.float32),
    )(block_e, mq_perm, wrow, memory, memt)

    c0 = out_perm[pos[0]]
    c1 = out_perm[pos[1]]
    out = pl.pallas_call(
        _tc3_body,
        grid=(N // T1,),
        in_specs=[
            pl.BlockSpec((T1, DM), lambda i: (i, 0)),
            pl.BlockSpec((T1, DM), lambda i: (i, 0)),
            pl.BlockSpec((DM, D), lambda i: (0, 0)),
            pl.BlockSpec((1, D), lambda i: (0, 0)),
        ],
        out_specs=pl.BlockSpec((T1, D), lambda i: (i, 0)),
        out_shape=jax.ShapeDtypeStruct((N, D), jnp.float32),
    )(c0, c1, outwt, outb2)

    return (out.reshape(B, S, D), loss.reshape(()))


# speedup vs baseline: 1.0234x; 1.0234x over previous
import dataclasses
import functools
import math

import jax
import jax.numpy as jnp
from jax import lax
from jax.experimental import pallas as pl
from jax.experimental.pallas import tpu as pltpu
from jax.experimental.pallas import tpu_sc as plsc

_BT = 256
_NC, _NS = 2, 16
_NW = _NC * _NS

def _sc_params():
    cp = pltpu.CompilerParams()
    if "needs_layout_passes" in pltpu.CompilerParams.__dataclass_fields__:
        cp = dataclasses.replace(cp, needs_layout_passes=False)
    return cp


def _tc1_body(hs_ref, q_ref, rwt_ref, inwt_ref, inb_ref,
              mqs_ref, ids_ref, tw_ref, cnt_ref, disp_ref, prob_ref,
              loss_ref, *, n_tokens, n_experts, scale):
    i = pl.program_id(0)
    nsteps = pl.num_programs(0)
    T = hs_ref.shape[0]
    E = n_experts

    logits = jnp.dot(hs_ref[...].astype(jnp.bfloat16),
                     rwt_ref[...].astype(jnp.bfloat16),
                     preferred_element_type=jnp.float32)
    w = jax.nn.softmax(logits, axis=-1)
    eids = jax.lax.broadcasted_iota(jnp.int32, (T, E), 1)
    w1 = jnp.max(w, axis=-1, keepdims=True)
    i1 = jnp.argmax(w, axis=-1).reshape(T, 1)
    wm = jnp.where(eids == i1, -jnp.inf, w)
    w2 = jnp.max(wm, axis=-1, keepdims=True)
    i2 = jnp.argmax(wm, axis=-1).reshape(T, 1)
    denom = w1 + w2 + 1e-8
    ids_ref[...] = jnp.concatenate([i1, i2], axis=1)
    tw_ref[...] = jnp.concatenate([w1 / denom, w2 / denom], axis=1)

    m1 = (eids == i1).astype(jnp.float32)
    m2 = (eids == i2).astype(jnp.float32)
    mboth = m1 + m2
    sub = [jnp.sum(mboth[c * 256:(c + 1) * 256], axis=0, keepdims=True)
           for c in range(T // 256)]
    cnt_ref[...] = jnp.concatenate(sub, axis=0).astype(
        jnp.int32).reshape(cnt_ref.shape)

    disp_part = jnp.sum(m1, axis=0, keepdims=True)
    prob_part = jnp.sum(w, axis=0, keepdims=True)

    @pl.when(i == 0)
    def _init():
        disp_ref[...] = jnp.zeros_like(disp_ref)
        prob_ref[...] = jnp.zeros_like(prob_ref)

    disp_ref[...] += disp_part
    prob_ref[...] += prob_part

    @pl.when(i == nsteps - 1)
    def _finalize():
        df = disp_ref[...] / n_tokens
        pf = prob_ref[...] / n_tokens
        loss_ref[...] = (E * jnp.sum(df * pf)).reshape(1, 1)

    log2e = 1.4426950408889634
    mq = jnp.dot(q_ref[...].astype(jnp.bfloat16),
                 inwt_ref[...].astype(jnp.bfloat16),
                 preferred_element_type=jnp.float32) + inb_ref[...]
    mqs_ref[...] = mq * (scale * log2e)


def _tc2_body(blocke_sref, mqp_ref, wrow_ref, mem_ref, memt_ref, outp_ref):
    b = pl.program_id(0)
    e = blocke_sref[b]
    mqb = mqp_ref[...].astype(jnp.bfloat16)
    attn = jnp.dot(mqb, memt_ref[e].astype(jnp.bfloat16),
                   preferred_element_type=jnp.float32)
    p = jnp.exp2(attn)
    s = jnp.sum(p, axis=-1, keepdims=True)
    eo = jnp.dot(p.astype(jnp.bfloat16), mem_ref[e].astype(jnp.bfloat16),
                 preferred_element_type=jnp.float32)
    wcol = wrow_ref[0, 0, :].reshape(outp_ref.shape[0], 1)
    outp_ref[...] = eo * (wcol / s)


def _tc3_body(c_ref, outwt_ref, outb_ref, out_ref):
    comb = c_ref[...].astype(jnp.bfloat16)
    out_ref[...] = jnp.dot(comb, outwt_ref[...].astype(jnp.bfloat16),
                           preferred_element_type=jnp.float32) + outb_ref[...]


def kernel(hidden_states, query, router_W, in_W, in_b, out_W, out_b, memory):
    B, S, D = hidden_states.shape
    E = router_W.shape[0]
    DM = in_W.shape[0]
    C = memory.shape[1]
    N = B * S
    P = 2 * N
    BT = _BT
    NB = P // BT + E
    NBPAD = ((NB + 15) // 16) * 16
    NPAD = NB * BT
    T1 = min(1024, N)
    CH = P // _NW
    GR = NPAD // _NW
    GCH = 96
    TT = N // _NW
    scale = 1.0 / math.sqrt(DM)

    hs2 = hidden_states.reshape(N, D)
    q2 = query.reshape(N, D)
    rwt = router_W.T
    inwt = in_W.T
    outwt = out_W.T
    memt = memory.transpose(0, 2, 1)
    inb2 = in_b.reshape(1, DM)
    outb2 = out_b.reshape(1, D)

    tc1 = functools.partial(_tc1_body, n_tokens=float(N), n_experts=E,
                            scale=scale)
    mqs, ids, tw, cnts, _, _, loss = pl.pallas_call(
        tc1,
        grid=(N // T1,),
        in_specs=[
            pl.BlockSpec((T1, D), lambda i: (i, 0)),
            pl.BlockSpec((T1, D), lambda i: (i, 0)),
            pl.BlockSpec((D, E), lambda i: (0, 0)),
            pl.BlockSpec((D, DM), lambda i: (0, 0)),
            pl.BlockSpec((1, DM), lambda i: (0, 0)),
        ],
        out_specs=[
            pl.BlockSpec((T1, DM), lambda i: (i, 0)),
            pl.BlockSpec((T1, 2), lambda i: (i, 0)),
            pl.BlockSpec((T1, 2), lambda i: (i, 0)),
            pl.BlockSpec((1, T1 // 256, E), lambda i: (i, 0, 0)),
            pl.BlockSpec((1, E), lambda i: (0, 0)),
            pl.BlockSpec((1, E), lambda i: (0, 0)),
            pl.BlockSpec((1, 1), lambda i: (0, 0)),
        ],
        out_shape=[
            jax.ShapeDtypeStruct((N, DM), jnp.float32),
            jax.ShapeDtypeStruct((N, 2), jnp.int32),
            jax.ShapeDtypeStruct((N, 2), jnp.float32),
            jax.ShapeDtypeStruct((N // T1, T1 // 256, E), jnp.int32),
            jax.ShapeDtypeStruct((1, E), jnp.float32),
            jax.ShapeDtypeStruct((1, E), jnp.float32),
            jax.ShapeDtypeStruct((1, 1), jnp.float32),
        ],
    )(hs2, q2, rwt, inwt, inb2)

    ids_flat = ids.reshape(P)
    tw_flat = tw.reshape(P)
    cnt2 = cnts.reshape(P // CH * E)

    mesh = plsc.VectorSubcoreMesh(core_axis_name="c", subcore_axis_name="s")

    @functools.partial(
        pl.kernel, mesh=mesh, compiler_params=_sc_params(),
        out_type=[
            jax.ShapeDtypeStruct((NPAD,), jnp.int32),
            jax.ShapeDtypeStruct((NPAD,), jnp.float32),
            jax.ShapeDtypeStruct((P,), jnp.int32),
            jax.ShapeDtypeStruct((NBPAD,), jnp.int32),
        ],
        scratch_types=[
            pltpu.VMEM((CH,), jnp.int32),
            pltpu.VMEM((CH,), jnp.float32),
            pltpu.VMEM((P // CH * E,), jnp.int32),
            pltpu.VMEM((CH // 128, 128), jnp.int32),
            pltpu.VMEM((CH // 128, 128), jnp.int32),
            pltpu.VMEM((CH // 128, 128), jnp.int32),
            pltpu.VMEM((CH // 128, 128), jnp.float32),
            pltpu.VMEM((NBPAD,), jnp.int32),
            pltpu.SemaphoreType.DMA,
        ],
    )
    def _sc_route(ids_hbm, tw_hbm, cnt_hbm, ptok_hbm, pw_hbm, pos_hbm,
                  be_hbm, idsv, twv, cntv, slotsv, tokv, destv, wv, bev,
                  sem):
        wid = lax.axis_index("s") * _NC + lax.axis_index("c")
        pltpu.sync_copy(ids_hbm.at[pl.ds(wid * CH, CH)], idsv)
        pltpu.sync_copy(tw_hbm.at[pl.ds(wid * CH, CH)], twv)
        pltpu.sync_copy(cnt_hbm, cntv)

        iota16 = lax.iota(jnp.int32, 16)
        half = (iota16 >= 8).astype(jnp.int32)
        tv = jnp.zeros((16,), jnp.int32)
        pv = jnp.zeros((16,), jnp.int32)
        for q in range(P // CH * E // 16):
            cv = cntv[pl.ds(q * 16, 16)]
            pred = ((2 * q + half) < wid).astype(jnp.int32)
            tv = tv + cv
            pv = pv + cv * pred
        total = [tv[e] + tv[e + 8] for e in range(E)]
        presum = [pv[e] + pv[e + 8] for e in range(E)]
        base = []
        run = jnp.int32(0)
        endblk = []
        for e in range(E):
            base.append(run)
            aligned = ((total[e] + BT - 1) // BT) * BT
            run = run + aligned
            endblk.append(run // BT)
        start = [base[e] + presum[e] for e in range(E)]

        for v in range(NBPAD // 16):
            bidx = lax.iota(jnp.int32, 16) + (v * 16)
            acc = jnp.zeros((16,), jnp.int32)
            for e in range(E):
                acc = acc + (bidx >= endblk[e]).astype(jnp.int32)
            bev[pl.ds(v * 16, 16)] = jnp.minimum(acc, E - 1)
        pltpu.sync_copy(bev, be_hbm)

        for j in range(CH // 16):
            v = idsv[pl.ds(j * 16, 16)]
            wvals = twv[pl.ds(j * 16, 16)]
            gp = lax.iota(jnp.int32, 16) + (wid * CH + j * 16)
            tok = gp >> 1
            kk = gp & 1
            dest = kk * N + tok
            slots = jnp.zeros((16,), jnp.int32)
            for e in range(E):
                m = v == e
                mi = m.astype(jnp.int32)
                r = plsc.cumsum(mi)
                slots = jnp.where(m, start[e] + r - 1, slots)
                start[e] = start[e] + jnp.sum(mi)
            c0, o0 = divmod(j, 8)
            sl = pl.ds(o0 * 16, 16)
            slotsv[c0, sl] = slots
            tokv[c0, sl] = tok
            destv[c0, sl] = dest
            wv[c0, sl] = wvals

        for c in range(CH // 128):
            pltpu.async_copy(tokv.at[c], ptok_hbm.at[slotsv.at[c]],
                             sem).wait()
            pltpu.async_copy(wv.at[c], pw_hbm.at[slotsv.at[c]], sem).wait()
            pltpu.async_copy(slotsv.at[c], pos_hbm.at[destv.at[c]],
                             sem).wait()

    perm_tok, perm_w, pos, blocke = _sc_route(ids_flat, tw_flat, cnt2)
    @functools.partial(
        pl.kernel, mesh=mesh, compiler_params=_sc_params(),
        out_type=jax.ShapeDtypeStruct((NPAD, DM), jnp.float32),
        scratch_types=[
            pltpu.VMEM((GCH,), jnp.int32),
            pltpu.VMEM((GCH, DM), jnp.float32),
            pltpu.SemaphoreType.DMA,
        ],
    )
    def _sc_gather(ptok_hbm, mqs_hbm, mqp_hbm, idxv, rowsv, sem):
        wid = lax.axis_index("s") * _NC + lax.axis_index("c")
        row0 = wid * GR
        for c in range(GR // GCH):
            pltpu.sync_copy(ptok_hbm.at[pl.ds(row0 + c * GCH, GCH)], idxv)
            for q in range(GCH // 16):
                sl = pl.ds(q * 16, 16)
                idxv[sl] = jnp.minimum(jnp.maximum(idxv[sl], 0), N - 1)
            pltpu.async_copy(mqs_hbm.at[idxv], rowsv, sem).wait()
            pltpu.sync_copy(rowsv, mqp_hbm.at[pl.ds(row0 + c * GCH, GCH)])

    mq_perm = _sc_gather(perm_tok, mqs)

    wrow = perm_w.reshape(NB, 1, BT)
    grid_spec = pltpu.PrefetchScalarGridSpec(
        num_scalar_prefetch=1,
        grid=(NB,),
        in_specs=[
            pl.BlockSpec((BT, DM), lambda b, sref: (b, 0)),
            pl.BlockSpec((1, 1, BT), lambda b, sref: (b, 0, 0)),
            pl.BlockSpec((E, C, DM), lambda b, sref: (0, 0, 0)),
            pl.BlockSpec((E, DM, C), lambda b, sref: (0, 0, 0)),
        ],
        out_specs=pl.BlockSpec((BT, DM), lambda b, sref: (b, 0)),
    )
    out_perm = pl.pallas_call(
        _tc2_body,
        grid_spec=grid_spec,
        out_shape=jax.ShapeDtypeStruct((NPAD, DM), jnp.float32),
    )(blocke[:NB], mq_perm, wrow, memory, memt)

    @functools.partial(
        pl.kernel, mesh=mesh, compiler_params=_sc_params(),
        out_type=jax.ShapeDtypeStruct((N, DM), jnp.float32),
        scratch_types=[
            pltpu.VMEM((2, 128), jnp.int32),
            pltpu.VMEM((128, DM), jnp.float32),
            pltpu.VMEM((128, DM), jnp.float32),
            pltpu.SemaphoreType.DMA,
        ],
    )
    def _sc_comb(pos_hbm, op_hbm, comb_hbm, posv, r0, r1, sem):
        wid = lax.axis_index("s") * _NC + lax.axis_index("c")
        t0 = wid * TT
        for c in range(TT // 128):
            tc0 = t0 + c * 128
            pltpu.sync_copy(pos_hbm.at[pl.ds(tc0, 128)], posv.at[0])
            pltpu.sync_copy(pos_hbm.at[pl.ds(N + tc0, 128)], posv.at[1])
            pltpu.async_copy(op_hbm.at[posv.at[0]], r0, sem).wait()
            pltpu.async_copy(op_hbm.at[posv.at[1]], r1, sem).wait()

            @pl.loop(0, 128)
            def _(r):
                for q in range(DM // 16):
                    sl = pl.ds(q * 16, 16)
                    r0[r, sl] = r0[r, sl] + r1[r, sl]

            pltpu.sync_copy(r0, comb_hbm.at[pl.ds(tc0, 128)])

    comb = _sc_comb(pos, out_perm)

    out = pl.pallas_call(
        _tc3_body,
        grid=(N // T1,),
        in_specs=[
            pl.BlockSpec((T1, DM), lambda i: (i, 0)),
            pl.BlockSpec((DM, D), lambda i: (0, 0)),
            pl.BlockSpec((1, D), lambda i: (0, 0)),
        ],
        out_specs=pl.BlockSpec((T1, D), lambda i: (i, 0)),
        out_shape=jax.ShapeDtypeStruct((N, D), jnp.float32),
    )(comb, outwt, outb2)

    return (out.reshape(B, S, D), loss.reshape(()))


# speedup vs baseline: 2.9876x; 2.9192x over previous
import functools
import math

import jax
import jax.numpy as jnp
from jax.experimental import pallas as pl


def _moe_body(hs_ref, q_ref, rwt_ref, inwt_ref, inb_ref, outwt_ref,
              outb_ref, mem_ref, memt_ref,
              out_ref, disp_ref, prob_ref, loss_ref,
              *, n_tokens, n_experts, scale):
    i = pl.program_id(0)
    nsteps = pl.num_programs(0)
    T = hs_ref.shape[0]
    E = n_experts

    logits = jnp.dot(hs_ref[...].astype(jnp.bfloat16),
                     rwt_ref[...].astype(jnp.bfloat16),
                     preferred_element_type=jnp.float32)
    w = jax.nn.softmax(logits, axis=-1)
    eids = jax.lax.broadcasted_iota(jnp.int32, (T, E), 1)
    w1 = jnp.max(w, axis=-1, keepdims=True)
    i1 = jnp.argmax(w, axis=-1).reshape(T, 1)
    wm = jnp.where(eids == i1, -jnp.inf, w)
    w2 = jnp.max(wm, axis=-1, keepdims=True)
    i2 = jnp.argmax(wm, axis=-1).reshape(T, 1)
    denom = w1 + w2 + 1e-8
    tw1 = w1 / denom
    tw2 = w2 / denom

    disp_part = jnp.sum((eids == i1).astype(jnp.float32), axis=0,
                        keepdims=True)
    prob_part = jnp.sum(w, axis=0, keepdims=True)

    @pl.when(i == 0)
    def _init():
        disp_ref[...] = jnp.zeros_like(disp_ref)
        prob_ref[...] = jnp.zeros_like(prob_ref)

    disp_ref[...] += disp_part
    prob_ref[...] += prob_part

    @pl.when(i == nsteps - 1)
    def _finalize():
        df = disp_ref[...] / n_tokens
        pf = prob_ref[...] / n_tokens
        loss_ref[...] = (E * jnp.sum(df * pf)).reshape(1, 1)

    mq = jnp.dot(q_ref[...].astype(jnp.bfloat16),
                 inwt_ref[...].astype(jnp.bfloat16),
                 preferred_element_type=jnp.float32) + inb_ref[...]
    log2e = 1.4426950408889634
    mqs = (mq * (scale * log2e)).astype(jnp.bfloat16)

    acc = jnp.zeros(mq.shape, jnp.float32)
    for e in range(E):
        attn = jnp.dot(mqs, memt_ref[e].astype(jnp.bfloat16),
                       preferred_element_type=jnp.float32)
        p = jnp.exp2(attn)
        s = jnp.sum(p, axis=-1, keepdims=True)
        eo = jnp.dot(p.astype(jnp.bfloat16),
                     mem_ref[e].astype(jnp.bfloat16),
                     preferred_element_type=jnp.float32)
        we = (tw1 * (i1 == e).astype(jnp.float32)
              + tw2 * (i2 == e).astype(jnp.float32)) / s
        acc = acc + we * eo

    out_ref[...] = jnp.dot(acc.astype(jnp.bfloat16),
                           outwt_ref[...].astype(jnp.bfloat16),
                           preferred_element_type=jnp.float32) + outb_ref[...]


def kernel(hidden_states, query, router_W, in_W, in_b, out_W, out_b, memory):
    B, S, D = hidden_states.shape
    E = router_W.shape[0]
    DM = in_W.shape[0]
    C = memory.shape[1]
    N = B * S
    T = 1024 if N % 1024 == 0 else N
    scale = 1.0 / math.sqrt(DM)

    hs2 = hidden_states.reshape(N, D)
    q2 = query.reshape(N, D)
    rwt = router_W.T
    inwt = in_W.T
    outwt = out_W.T
    memt = memory.transpose(0, 2, 1)
    inb2 = in_b.reshape(1, DM)
    outb2 = out_b.reshape(1, D)

    grid = (N // T,)
    body = functools.partial(_moe_body, n_tokens=float(N), n_experts=E,
                             scale=scale)
    out, _, _, loss = pl.pallas_call(
        body,
        grid=grid,
        in_specs=[
            pl.BlockSpec((T, D), lambda i: (i, 0)),
            pl.BlockSpec((T, D), lambda i: (i, 0)),
            pl.BlockSpec((D, E), lambda i: (0, 0)),
            pl.BlockSpec((D, DM), lambda i: (0, 0)),
            pl.BlockSpec((1, DM), lambda i: (0, 0)),
            pl.BlockSpec((DM, D), lambda i: (0, 0)),
            pl.BlockSpec((1, D), lambda i: (0, 0)),
            pl.BlockSpec((E, C, DM), lambda i: (0, 0, 0)),
            pl.BlockSpec((E, DM, C), lambda i: (0, 0, 0)),
        ],
        out_specs=[
            pl.BlockSpec((T, D), lambda i: (i, 0)),
            pl.BlockSpec((1, E), lambda i: (0, 0)),
            pl.BlockSpec((1, E), lambda i: (0, 0)),
            pl.BlockSpec((1, 1), lambda i: (0, 0)),
        ],
        out_shape=[
            jax.ShapeDtypeStruct((N, D), jnp.float32),
            jax.ShapeDtypeStruct((1, E), jnp.float32),
            jax.ShapeDtypeStruct((1, E), jnp.float32),
            jax.ShapeDtypeStruct((1, 1), jnp.float32),
        ],
    )(hs2, q2, rwt, inwt, inb2, outwt, outb2, memory, memt)

    return (out.reshape(B, S, D), loss.reshape(()))
